# baseline (speedup 1.0000x reference)
import jax
import jax.numpy as jnp
from jax.experimental import pallas as pl

N_FREQ = 10
HID = 128
HEADS = 2
DH = 64
RANK = 128
INV_SQRT_D = 0.125


def _iota(shape, dim):
    return jax.lax.broadcasted_iota(jnp.int32, shape, dim)


def _mm(a, b, prec=jax.lax.Precision.DEFAULT):
    return jax.lax.dot_general(a, b, (((1,), (0,)), ((), ())),
                               precision=prec,
                               preferred_element_type=jnp.float32)


def _mmt(a, b, prec=jax.lax.Precision.DEFAULT):
    return jax.lax.dot_general(a, b, (((1,), (1,)), ((), ())),
                               precision=prec,
                               preferred_element_type=jnp.float32)


_TWO_PI_HI = 6.28125
_TWO_PI_LO = 0.0019353071795864769
_INV_TWO_PI = 0.15915494309189535


def _sincos_reduced(x):
    k = jnp.round(x * _INV_TWO_PI)
    r = (x - k * _TWO_PI_HI) - k * _TWO_PI_LO
    return jnp.sin(r), jnp.cos(r)


def _body(scene_ref, spos_ref, grip_ref, gpos_ref, lemb_ref, lw_ref, lb_ref,
          lee_ref, wq_ref, wk_ref, wv_ref, we_ref, lng_ref, lnb_ref,
          out_ref):
    f32 = jnp.float32
    B = lemb_ref.shape[0]
    NS = scene_ref.shape[0] // B
    NG = grip_ref.shape[0] // B
    L = wq_ref.shape[0]

    zrow = jnp.zeros((1, HID), f32)

    def build_C(l, r):
        w = we_ref[l, r]
        w63 = w[0:63, :] + w[63:126, :]
        return jnp.concatenate([w63, zrow, w63, zrow], axis=0)

    Cmats = [[build_C(l, r) for r in range(3)] for l in range(L)]

    r3 = _iota((3, 32), 0)
    c3 = _iota((3, 32), 1)
    freq = jnp.exp2((c3 // 3).astype(f32))
    G = jnp.where((c3 % 3 == r3) & (c3 < 30), freq, 0.0)

    rm = _iota((32, 128), 0)
    cm = _iota((32, 128), 1)
    ok = rm < 30
    AUs = jnp.where(ok & ((cm == rm + 3) | (cm == rm + 97)), 1.0, 0.0)
    AUc = jnp.where(ok & ((cm == rm + 33) | (cm == rm + 67)), 1.0, 0.0)
    AVs = (jnp.where(ok & (cm == rm + 97), 1.0, 0.0)
           - jnp.where(ok & (cm == rm + 67), 1.0, 0.0))
    AVc = jnp.where(ok & ((cm == rm + 3) | (cm == rm + 33)), 1.0, 0.0)
    rp = _iota((3, 128), 0)
    cp = _iota((3, 128), 1)
    AUp = jnp.where(cp == rp, 1.0, 0.0)
    AVp = -jnp.where(cp == rp + 64, 1.0, 0.0)
    lane = _iota((1, 128), 1)
    onesU = jnp.where((lane >= 64) & (lane < 67), 1.0, 0.0)
    onesV = jnp.where(lane < 3, 1.0, 0.0)

    def posfeat(pos):
        hi = jax.lax.Precision.HIGHEST
        P = _mm(pos, G, hi)
        S, Cc = _sincos_reduced(P)
        U = _mm(pos, AUp, hi) + _mm(S, AUs, hi) + _mm(Cc, AUc, hi) + onesU
        V = _mm(pos, AVp, hi) + _mm(S, AVs, hi) + _mm(Cc, AVc, hi) + onesV
        return U, V

    U_s, V_s = posfeat(spos_ref[...])
    U_g, V_g = posfeat(gpos_ref[...])

    lang = _mm(lemb_ref[...], lw_ref[...]) + lb_ref[...]

    oh_s = jnp.where(_iota((B * NS, B), 1) == _iota((B * NS, B), 0) // NS,
                     1.0, 0.0)
    oh_g = jnp.where(_iota((B * NG, B), 1) == _iota((B * NG, B), 0) // NG,
                     1.0, 0.0)

    def ln(x, g, b):
        mu = jnp.mean(x, axis=1, keepdims=True)
        d = x - mu
        var = jnp.mean(d * d, axis=1, keepdims=True)
        return d * jax.lax.rsqrt(var + 1e-5) * g + b

    mask_ss = jnp.where(_iota((B * NS, B * NS), 0) // NS
                        == _iota((B * NS, B * NS), 1) // NS, 0.0, -1e30)
    mask_gg = jnp.where(_iota((B * NG, B * NG), 0) // NG
                        == _iota((B * NG, B * NG), 1) // NG, 0.0, -1e30)
    mask_sg = jnp.where(_iota((B * NG, B * NS), 0) // NG
                        == _iota((B * NG, B * NS), 1) // NS, 0.0, -1e30)

    def conv_bd(xd, xs, U, V, wq, wk, wv, Cm, mask):
        q = _mm(xd, wq) * INV_SQRT_D
        k = _mm(xs, wk)
        v = _mm(xs, wv)
        head_outs = []
        for h in range(HEADS):
            sl = slice(h * DH, (h + 1) * DH)
            qh = q[:, sl]
            Ch = Cm[:, sl]
            Qe = jnp.concatenate([U * _mmt(qh, Ch), qh], axis=1)
            Ke = jnp.concatenate([V, k[:, sl]], axis=1)
            s = _mmt(Qe, Ke) + mask
            amax = jnp.max(s, axis=1, keepdims=True)
            p = jnp.exp(s - amax)
            den = jnp.sum(p, axis=1, keepdims=True)
            agg = _mm(p, jnp.concatenate([V, v[:, sl]], axis=1))
            o = (agg[:, RANK:] + _mm(U * agg[:, :RANK], Ch)) / (den + 1e-16)
            head_outs.append(o)
        return jnp.concatenate(head_outs, axis=1)

    def conv(xd, xs, U, V, wq, wk, wv, Cm, nd, ns):
        q = _mm(xd, wq) * INV_SQRT_D
        k = _mm(xs, wk)
        v = _mm(xs, wv)
        blocks = []
        for h in range(HEADS):
            sl = slice(h * DH, (h + 1) * DH)
            qh = q[:, sl]
            Ch = Cm[:, sl]
            Qe = jnp.concatenate([U * _mmt(qh, Ch), qh], axis=1)
            for b in range(B):
                dsl = slice(b * nd, (b + 1) * nd)
                ssl = slice(b * ns, (b + 1) * ns)
                Ke = jnp.concatenate([V[ssl], k[ssl, sl]], axis=1)
                blocks.append((h, b, Ch, _mmt(Qe[dsl], Ke)))
        soft = []
        for h, b, Ch, s in blocks:
            amax = jnp.max(s, axis=1, keepdims=True)
            p = jnp.exp(s - amax)
            den = jnp.sum(p, axis=1, keepdims=True)
            soft.append((h, b, Ch, p, den))
        outs = [[None] * B for _ in range(HEADS)]
        for h, b, Ch, p, den in soft:
            sl = slice(h * DH, (h + 1) * DH)
            dsl = slice(b * nd, (b + 1) * nd)
            ssl = slice(b * ns, (b + 1) * ns)
            agg = _mm(p, jnp.concatenate([V[ssl], v[ssl, sl]], axis=1))
            outs[h][b] = (agg[:, RANK:] + _mm(U[dsl] * agg[:, :RANK], Ch)) \
                / (den + 1e-16)
        return jnp.concatenate(
            [jnp.concatenate(outs[h], axis=0) for h in range(HEADS)], axis=1)

    scene = scene_ref[...]
    grip = grip_ref[...]
    for l in range(L):
        lmsg_s = _mm(lang, wv_ref[l, 3]) + _mm(lee_ref[...], we_ref[l, 3])
        lmsg_g = _mm(lang, wv_ref[l, 4]) + _mm(lee_ref[...], we_ref[l, 4])
        m_s = (conv_bd(scene, scene, U_s, V_s, wq_ref[l, 0], wk_ref[l, 0],
                       wv_ref[l, 0], Cmats[l][0], mask_ss)
               + _mm(oh_s, lmsg_s))
        m_g = (conv_bd(grip, scene, U_g, V_s, wq_ref[l, 1], wk_ref[l, 1],
                       wv_ref[l, 1], Cmats[l][1], mask_sg)
               + conv_bd(grip, grip, U_g, V_g, wq_ref[l, 2], wk_ref[l, 2],
                         wv_ref[l, 2], Cmats[l][2], mask_gg)
               + _mm(oh_g, lmsg_g))
        scene = ln(scene + m_s, lng_ref[l, 0], lnb_ref[l, 0])
        grip = ln(grip + m_g, lng_ref[l, 1], lnb_ref[l, 1])

    out_ref[...] = grip.reshape(B, NG, HID)


def kernel(scene_x, scene_pos, gripper_x, gripper_pos, lang_emb, lang_W,
           lang_b, lang_edge_emb, Wq, Wk, Wv, We, ln_g, ln_b):
    B, NS, H = scene_x.shape
    NG = gripper_x.shape[1]
    L = Wq.shape[0]
    f32 = jnp.float32

    return pl.pallas_call(
        _body,
        out_shape=jax.ShapeDtypeStruct((B, NG, H), f32),
    )(scene_x.reshape(B * NS, H), scene_pos.reshape(B * NS, 3),
      gripper_x.reshape(B * NG, H), gripper_pos.reshape(B * NG, 3),
      lang_emb, lang_W, lang_b.reshape(1, H), lang_edge_emb,
      Wq, Wk, Wv, We,
      ln_g.reshape(L, 2, 1, H), ln_b.reshape(L, 2, 1, H))


# speedup vs baseline: 1.0403x; 1.0403x over previous
import jax
import jax.numpy as jnp
from jax.experimental import pallas as pl

N_FREQ = 10
HID = 128
HEADS = 2
DH = 64
RANK = 128
INV_SQRT_D = 0.125


def _iota(shape, dim):
    return jax.lax.broadcasted_iota(jnp.int32, shape, dim)


def _mm(a, b, prec=jax.lax.Precision.DEFAULT):
    return jax.lax.dot_general(a, b, (((1,), (0,)), ((), ())),
                               precision=prec,
                               preferred_element_type=jnp.float32)


def _mmt(a, b, prec=jax.lax.Precision.DEFAULT):
    return jax.lax.dot_general(a, b, (((1,), (1,)), ((), ())),
                               precision=prec,
                               preferred_element_type=jnp.float32)


_TWO_PI_HI = 6.28125
_TWO_PI_LO = 0.0019353071795864769
_INV_TWO_PI = 0.15915494309189535


def _sincos_reduced(x):
    k = jnp.round(x * _INV_TWO_PI)
    r = (x - k * _TWO_PI_HI) - k * _TWO_PI_LO
    return jnp.sin(r), jnp.cos(r)


def _body(scene_ref, spos_ref, grip_ref, gpos_ref, lemb_ref, lw_ref, lb_ref,
          lee_ref, wq_ref, wk_ref, wv_ref, we_ref, lng_ref, lnb_ref,
          out_ref):
    f32 = jnp.float32
    B = lemb_ref.shape[0]
    NS = scene_ref.shape[0] // B
    NG = grip_ref.shape[0] // B
    L = wq_ref.shape[0]

    zrow = jnp.zeros((1, HID), f32)

    def build_C(l, r):
        w = we_ref[l, r]
        w63 = w[0:63, :] + w[63:126, :]
        return jnp.concatenate([w63, zrow, w63, zrow], axis=0)

    Cmats = [[build_C(l, r) for r in range(3)] for l in range(L)]

    r3 = _iota((3, 32), 0)
    c3 = _iota((3, 32), 1)
    freq = jnp.exp2((c3 // 3).astype(f32))
    G = jnp.where((c3 % 3 == r3) & (c3 < 30), freq, 0.0)

    rm = _iota((32, 128), 0)
    cm = _iota((32, 128), 1)
    ok = rm < 30
    AUs = jnp.where(ok & ((cm == rm + 3) | (cm == rm + 97)), 1.0, 0.0)
    AUc = jnp.where(ok & ((cm == rm + 33) | (cm == rm + 67)), 1.0, 0.0)
    AVs = (jnp.where(ok & (cm == rm + 97), 1.0, 0.0)
           - jnp.where(ok & (cm == rm + 67), 1.0, 0.0))
    AVc = jnp.where(ok & ((cm == rm + 3) | (cm == rm + 33)), 1.0, 0.0)
    rp = _iota((3, 128), 0)
    cp = _iota((3, 128), 1)
    AUp = jnp.where(cp == rp, 1.0, 0.0)
    AVp = -jnp.where(cp == rp + 64, 1.0, 0.0)
    lane = _iota((1, 128), 1)
    onesU = jnp.where((lane >= 64) & (lane < 67), 1.0, 0.0)
    onesV = jnp.where(lane < 3, 1.0, 0.0)

    def posfeat(pos):
        hi = jax.lax.Precision.HIGHEST
        P = _mm(pos, G, hi)
        S, Cc = _sincos_reduced(P)
        U = _mm(pos, AUp, hi) + _mm(S, AUs, hi) + _mm(Cc, AUc, hi) + onesU
        V = _mm(pos, AVp, hi) + _mm(S, AVs, hi) + _mm(Cc, AVc, hi) + onesV
        return U, V

    U_s, V_s = posfeat(spos_ref[...])
    U_g, V_g = posfeat(gpos_ref[...])

    lang = _mm(lemb_ref[...], lw_ref[...]) + lb_ref[...]

    oh_s = jnp.where(_iota((B * NS, B), 1) == _iota((B * NS, B), 0) // NS,
                     1.0, 0.0)
    oh_g = jnp.where(_iota((B * NG, B), 1) == _iota((B * NG, B), 0) // NG,
                     1.0, 0.0)

    def ln(x, g, b):
        mu = jnp.mean(x, axis=1, keepdims=True)
        d = x - mu
        var = jnp.mean(d * d, axis=1, keepdims=True)
        return d * jax.lax.rsqrt(var + 1e-5) * g + b

    mask_gg = jnp.where(_iota((B * NG, B * NG), 0) // NG
                        == _iota((B * NG, B * NG), 1) // NG, 0.0, -1e30)
    mask_sg = jnp.where(_iota((B * NG, B * NS), 0) // NG
                        == _iota((B * NG, B * NS), 1) // NS, 0.0, -1e30)

    def conv_bd(q, k, v, U, V, Cm, mask):
        head_outs = []
        for h in range(HEADS):
            sl = slice(h * DH, (h + 1) * DH)
            qh = q[:, sl]
            Ch = Cm[:, sl]
            Qe = jnp.concatenate([U * _mmt(qh, Ch), qh], axis=1)
            Ke = jnp.concatenate([V, k[:, sl]], axis=1)
            s = _mmt(Qe, Ke) + mask
            amax = jnp.max(s, axis=1, keepdims=True)
            p = jnp.exp(s - amax)
            den = jnp.sum(p, axis=1, keepdims=True)
            agg = _mm(p, jnp.concatenate([V, v[:, sl]], axis=1))
            o = (agg[:, RANK:] + _mm(U * agg[:, :RANK], Ch)) / (den + 1e-16)
            head_outs.append(o)
        return jnp.concatenate(head_outs, axis=1)

    def conv(q, k, v, U, V, Cm, nd, ns):
        blocks = []
        for h in range(HEADS):
            sl = slice(h * DH, (h + 1) * DH)
            qh = q[:, sl]
            Ch = Cm[:, sl]
            Qe = jnp.concatenate([U * _mmt(qh, Ch), qh], axis=1)
            for b in range(B):
                dsl = slice(b * nd, (b + 1) * nd)
                ssl = slice(b * ns, (b + 1) * ns)
                Ke = jnp.concatenate([V[ssl], k[ssl, sl]], axis=1)
                blocks.append((h, b, Ch, _mmt(Qe[dsl], Ke)))
        soft = []
        for h, b, Ch, s in blocks:
            amax = jnp.max(s, axis=1, keepdims=True)
            p = jnp.exp(s - amax)
            den = jnp.sum(p, axis=1, keepdims=True)
            soft.append((h, b, Ch, p, den))
        outs = [[None] * B for _ in range(HEADS)]
        for h, b, Ch, p, den in soft:
            sl = slice(h * DH, (h + 1) * DH)
            dsl = slice(b * nd, (b + 1) * nd)
            ssl = slice(b * ns, (b + 1) * ns)
            agg = _mm(p, jnp.concatenate([V[ssl], v[ssl, sl]], axis=1))
            outs[h][b] = (agg[:, RANK:] + _mm(U[dsl] * agg[:, :RANK], Ch)) \
                / (den + 1e-16)
        return jnp.concatenate(
            [jnp.concatenate(outs[h], axis=0) for h in range(HEADS)], axis=1)

    scene = scene_ref[...]
    grip = grip_ref[...]
    for l in range(L):
        sp = _mm(scene, jnp.concatenate(
            [wq_ref[l, 0], wk_ref[l, 0], wv_ref[l, 0],
             wk_ref[l, 1], wv_ref[l, 1]], axis=1))
        gp = _mm(grip, jnp.concatenate(
            [wq_ref[l, 1], wq_ref[l, 2], wk_ref[l, 2],
             wv_ref[l, 2]], axis=1))
        lmsg = (_mm(lang, jnp.concatenate(
                    [wv_ref[l, 3], wv_ref[l, 4]], axis=1))
                + _mm(lee_ref[...], jnp.concatenate(
                    [we_ref[l, 3], we_ref[l, 4]], axis=1)))
        m_s = (conv(sp[:, :HID] * INV_SQRT_D, sp[:, HID:2 * HID],
                    sp[:, 2 * HID:3 * HID], U_s, V_s, Cmats[l][0], NS, NS)
               + _mm(oh_s, lmsg[:, :HID]))
        m_g = (conv_bd(gp[:, :HID] * INV_SQRT_D, sp[:, 3 * HID:4 * HID],
                       sp[:, 4 * HID:], U_g, V_s, Cmats[l][1], mask_sg)
               + conv_bd(gp[:, HID:2 * HID] * INV_SQRT_D,
                         gp[:, 2 * HID:3 * HID], gp[:, 3 * HID:],
                         U_g, V_g, Cmats[l][2], mask_gg)
               + _mm(oh_g, lmsg[:, HID:]))
        scene = ln(scene + m_s, lng_ref[l, 0], lnb_ref[l, 0])
        grip = ln(grip + m_g, lng_ref[l, 1], lnb_ref[l, 1])

    out_ref[...] = grip.reshape(B, NG, HID)


def kernel(scene_x, scene_pos, gripper_x, gripper_pos, lang_emb, lang_W,
           lang_b, lang_edge_emb, Wq, Wk, Wv, We, ln_g, ln_b):
    B, NS, H = scene_x.shape
    NG = gripper_x.shape[1]
    L = Wq.shape[0]
    f32 = jnp.float32

    return pl.pallas_call(
        _body,
        out_shape=jax.ShapeDtypeStruct((B, NG, H), f32),
    )(scene_x.reshape(B * NS, H), scene_pos.reshape(B * NS, 3),
      gripper_x.reshape(B * NG, H), gripper_pos.reshape(B * NG, 3),
      lang_emb, lang_W, lang_b.reshape(1, H), lang_edge_emb,
      Wq, Wk, Wv, We,
      ln_g.reshape(L, 2, 1, H), ln_b.reshape(L, 2, 1, H))


# speedup vs baseline: 1.0724x; 1.0308x over previous
import jax
import jax.numpy as jnp
from jax.experimental import pallas as pl

N_FREQ = 10
HID = 128
HEADS = 2
DH = 64
RANK = 128
INV_SQRT_D = 0.125


def _iota(shape, dim):
    return jax.lax.broadcasted_iota(jnp.int32, shape, dim)


def _mm(a, b, prec=jax.lax.Precision.DEFAULT):
    return jax.lax.dot_general(a, b, (((1,), (0,)), ((), ())),
                               precision=prec,
                               preferred_element_type=jnp.float32)


def _mmt(a, b, prec=jax.lax.Precision.DEFAULT):
    return jax.lax.dot_general(a, b, (((1,), (1,)), ((), ())),
                               precision=prec,
                               preferred_element_type=jnp.float32)


_TWO_PI_HI = 6.28125
_TWO_PI_LO = 0.0019353071795864769
_INV_TWO_PI = 0.15915494309189535


def _sincos_reduced(x):
    k = jnp.round(x * _INV_TWO_PI)
    r = (x - k * _TWO_PI_HI) - k * _TWO_PI_LO
    return jnp.sin(r), jnp.cos(r)


def _body(scene_ref, spos_ref, grip_ref, gpos_ref, lemb_ref, lw_ref, lb_ref,
          lee_ref, wq_ref, wk_ref, wv_ref, we_ref, lng_ref, lnb_ref,
          out_ref):
    f32 = jnp.float32
    B = lemb_ref.shape[0]
    NS = scene_ref.shape[0] // B
    NG = grip_ref.shape[0] // B
    L = wq_ref.shape[0]

    zrow = jnp.zeros((1, HID), f32)

    def build_C(l, r):
        w = we_ref[l, r]
        w63 = w[0:63, :] + w[63:126, :]
        return jnp.concatenate([w63, zrow, w63, zrow], axis=0)

    Cmats = [[build_C(l, r) for r in range(3)] for l in range(L)]

    r3 = _iota((3, 32), 0)
    c3 = _iota((3, 32), 1)
    freq = jnp.exp2((c3 // 3).astype(f32))
    G = jnp.where((c3 % 3 == r3) & (c3 < 30), freq, 0.0)

    rm = _iota((32, 128), 0)
    cm = _iota((32, 128), 1)
    ok = rm < 30
    AUs = jnp.where(ok & ((cm == rm + 3) | (cm == rm + 97)), 1.0, 0.0)
    AUc = jnp.where(ok & ((cm == rm + 33) | (cm == rm + 67)), 1.0, 0.0)
    AVs = (jnp.where(ok & (cm == rm + 97), 1.0, 0.0)
           - jnp.where(ok & (cm == rm + 67), 1.0, 0.0))
    AVc = jnp.where(ok & ((cm == rm + 3) | (cm == rm + 33)), 1.0, 0.0)
    rp = _iota((3, 128), 0)
    cp = _iota((3, 128), 1)
    AUp = jnp.where(cp == rp, 1.0, 0.0)
    AVp = -jnp.where(cp == rp + 64, 1.0, 0.0)
    lane = _iota((1, 128), 1)
    onesU = jnp.where((lane >= 64) & (lane < 67), 1.0, 0.0)
    onesV = jnp.where(lane < 3, 1.0, 0.0)

    def posfeat(pos):
        hi = jax.lax.Precision.HIGHEST
        P = _mm(pos, G, hi)
        S, Cc = _sincos_reduced(P)
        U = _mm(pos, AUp, hi) + _mm(S, AUs, hi) + _mm(Cc, AUc, hi) + onesU
        V = _mm(pos, AVp, hi) + _mm(S, AVs, hi) + _mm(Cc, AVc, hi) + onesV
        return U, V

    U_s, V_s = posfeat(spos_ref[...])
    U_g, V_g = posfeat(gpos_ref[...])

    lang = _mm(lemb_ref[...], lw_ref[...]) + lb_ref[...]

    oh_s = jnp.where(_iota((B * NS, B), 1) == _iota((B * NS, B), 0) // NS,
                     1.0, 0.0)
    oh_g = jnp.where(_iota((B * NG, B), 1) == _iota((B * NG, B), 0) // NG,
                     1.0, 0.0)

    def ln(x, g, b):
        mu = jnp.mean(x, axis=1, keepdims=True)
        d = x - mu
        var = jnp.mean(d * d, axis=1, keepdims=True)
        return d * jax.lax.rsqrt(var + 1e-5) * g + b

    mask_gg = jnp.where(_iota((B * NG, B * NG), 0) // NG
                        == _iota((B * NG, B * NG), 1) // NG, 0.0, -1e30)
    mask_sg = jnp.where(_iota((B * NG, B * NS), 0) // NG
                        == _iota((B * NG, B * NS), 1) // NS, 0.0, -1e30)

    def conv_bd(xd, xs, U, V, wq, wk, wv, Cm, mask):
        q = _mm(xd, wq) * INV_SQRT_D
        k = _mm(xs, wk)
        v = _mm(xs, wv)
        head_outs = []
        for h in range(HEADS):
            sl = slice(h * DH, (h + 1) * DH)
            qh = q[:, sl]
            Ch = Cm[:, sl]
            Qe = jnp.concatenate([U * _mmt(qh, Ch), qh], axis=1)
            Ke = jnp.concatenate([V, k[:, sl]], axis=1)
            s = _mmt(Qe, Ke) + mask
            amax = jnp.max(s, axis=1, keepdims=True)
            p = jnp.exp(s - amax)
            den = jnp.sum(p, axis=1, keepdims=True)
            agg = _mm(p, jnp.concatenate([V, v[:, sl]], axis=1))
            o = (agg[:, RANK:] + _mm(U * agg[:, :RANK], Ch)) / (den + 1e-16)
            head_outs.append(o)
        return jnp.concatenate(head_outs, axis=1)

    def conv(xd, xs, U, V, wq, wk, wv, Cm, nd, ns):
        q = _mm(xd, wq) * INV_SQRT_D
        k = _mm(xs, wk)
        v = _mm(xs, wv)
        blocks = []
        for h in range(HEADS):
            sl = slice(h * DH, (h + 1) * DH)
            qh = q[:, sl]
            Ch = Cm[:, sl]
            Qe = jnp.concatenate([U * _mmt(qh, Ch), qh], axis=1)
            for b in range(B):
                dsl = slice(b * nd, (b + 1) * nd)
                ssl = slice(b * ns, (b + 1) * ns)
                Ke = jnp.concatenate([V[ssl], k[ssl, sl]], axis=1)
                blocks.append((h, b, Ch, _mmt(Qe[dsl], Ke)))
        soft = []
        for h, b, Ch, s in blocks:
            amax = jnp.max(s, axis=1, keepdims=True)
            p = jnp.exp(s - amax)
            den = jnp.sum(p, axis=1, keepdims=True)
            soft.append((h, b, Ch, p, den))
        outs = [[None] * B for _ in range(HEADS)]
        for h, b, Ch, p, den in soft:
            sl = slice(h * DH, (h + 1) * DH)
            dsl = slice(b * nd, (b + 1) * nd)
            ssl = slice(b * ns, (b + 1) * ns)
            agg = _mm(p, jnp.concatenate([V[ssl], v[ssl, sl]], axis=1))
            outs[h][b] = (agg[:, RANK:] + _mm(U[dsl] * agg[:, :RANK], Ch)) \
                / (den + 1e-16)
        return jnp.concatenate(
            [jnp.concatenate(outs[h], axis=0) for h in range(HEADS)], axis=1)

    scene = scene_ref[...]
    grip = grip_ref[...]
    for l in range(L):
        lmsg_s = _mm(lang, wv_ref[l, 3]) + _mm(lee_ref[...], we_ref[l, 3])
        lmsg_g = _mm(lang, wv_ref[l, 4]) + _mm(lee_ref[...], we_ref[l, 4])
        m_s = (conv(scene, scene, U_s, V_s, wq_ref[l, 0], wk_ref[l, 0],
                    wv_ref[l, 0], Cmats[l][0], NS, NS)
               + _mm(oh_s, lmsg_s))
        m_g = (conv_bd(grip, scene, U_g, V_s, wq_ref[l, 1], wk_ref[l, 1],
                       wv_ref[l, 1], Cmats[l][1], mask_sg)
               + conv_bd(grip, grip, U_g, V_g, wq_ref[l, 2], wk_ref[l, 2],
                         wv_ref[l, 2], Cmats[l][2], mask_gg)
               + _mm(oh_g, lmsg_g))
        scene = ln(scene + m_s, lng_ref[l, 0], lnb_ref[l, 0])
        grip = ln(grip + m_g, lng_ref[l, 1], lnb_ref[l, 1])

    out_ref[...] = grip.reshape(B, NG, HID)


def kernel(scene_x, scene_pos, gripper_x, gripper_pos, lang_emb, lang_W,
           lang_b, lang_edge_emb, Wq, Wk, Wv, We, ln_g, ln_b):
    B, NS, H = scene_x.shape
    NG = gripper_x.shape[1]
    L = Wq.shape[0]
    f32 = jnp.float32

    return pl.pallas_call(
        _body,
        out_shape=jax.ShapeDtypeStruct((B, NG, H), f32),
    )(scene_x.reshape(B * NS, H), scene_pos.reshape(B * NS, 3),
      gripper_x.reshape(B * NG, H), gripper_pos.reshape(B * NG, 3),
      lang_emb, lang_W, lang_b.reshape(1, H), lang_edge_emb,
      Wq, Wk, Wv, We,
      ln_g.reshape(L, 2, 1, H), ln_b.reshape(L, 2, 1, H))
